# baseline (speedup 1.0000x reference)
import functools

import jax
import jax.numpy as jnp
from jax import lax
from jax.experimental import pallas as pl
from jax.experimental.pallas import tpu as pltpu
from jax.experimental.pallas import tpu_sc as plsc

_ROWS = 1_000_000
_DIM = 64
_LANES = 16
_WORKERS = 32
_CHUNK_ROWS = 400
_NCHUNKS = _ROWS // _CHUNK_ROWS
_GROUPS = _CHUNK_ROWS // _LANES


def _rsqrt16(t):
    i = plsc.bitcast(t, jnp.int32)
    i = jnp.full((_LANES,), 0x5F3759DF, jnp.int32) - (i >> 1)
    y = plsc.bitcast(i, jnp.float32)
    half_t = t * 0.5
    for _ in range(3):
        y = y * (1.5 - half_t * y * y)
    return y


def _normalize_group(xbuf, sbuf, ybuf, g):
    iota = lax.broadcasted_iota(jnp.int32, (_LANES,), 0)
    base = g * _LANES
    for r in range(_LANES):
        row = base + r
        acc = None
        for j in range(_DIM // _LANES):
            v = xbuf[row, pl.ds(j * _LANES, _LANES)]
            sq = v * v
            acc = sq if acc is None else acc + sq
        sbuf[r, :] = acc
    tot = None
    for c in range(_LANES):
        col = plsc.load_gather(sbuf, [iota, jnp.full((_LANES,), c, jnp.int32)])
        tot = col if tot is None else tot + col
    ybuf[...] = _rsqrt16(tot)
    for r in range(_LANES):
        row = base + r
        scale = plsc.load_gather(ybuf, [jnp.full((_LANES,), r, jnp.int32)])
        for j in range(_DIM // _LANES):
            sl = pl.ds(j * _LANES, _LANES)
            xbuf[row, sl] = xbuf[row, sl] * scale


def _sc_body(w_hbm, o_hbm, xbuf, sbuf, ybuf):
    wid = lax.axis_index("s") * 2 + lax.axis_index("c")
    nit = (_NCHUNKS - 1 - wid) // _WORKERS + 1

    def chunk_step(i, carry):
        k = wid + i * _WORKERS
        row0 = k * _CHUNK_ROWS
        pltpu.sync_copy(w_hbm.at[pl.ds(row0, _CHUNK_ROWS)], xbuf)

        def group_step(g, c2):
            _normalize_group(xbuf, sbuf, ybuf, g)
            return c2

        lax.fori_loop(0, _GROUPS, group_step, 0, unroll=False)
        pltpu.sync_copy(xbuf, o_hbm.at[pl.ds(row0, _CHUNK_ROWS)])
        return carry

    lax.fori_loop(0, nit, chunk_step, 0, unroll=False)


def kernel(weight):
    mesh = plsc.VectorSubcoreMesh(core_axis_name="c", subcore_axis_name="s")
    run = functools.partial(
        pl.kernel,
        mesh=mesh,
        out_type=jax.ShapeDtypeStruct((_ROWS, _DIM), jnp.float32),
        scratch_types=[
            pltpu.VMEM((_CHUNK_ROWS, _DIM), jnp.float32),
            pltpu.VMEM((_LANES, _LANES), jnp.float32),
            pltpu.VMEM((_LANES,), jnp.float32),
        ],
        compiler_params=pltpu.CompilerParams(needs_layout_passes=False),
    )(_sc_body)
    return run(weight)
